# Initial kernel scaffold; baseline (speedup 1.0000x reference)
import jax
import jax.numpy as jnp
from jax.experimental import pallas as pl


def kernel(features, edge_index, W1, al1, ar1, b1, W2, al2, ar2, b2, fcW, fcb):
    raise NotImplementedError("write your pallas kernel here")


# speedup vs baseline: 27.0618x; 27.0618x over previous
import functools

import jax
import jax.numpy as jnp
from jax import lax
from jax.experimental import pallas as pl
from jax.experimental.pallas import tpu as pltpu
from jax.experimental.pallas import tpu_sc as plsc

N = 10000
E = 320000
D = 128
BLK = 2000
GRID = N // BLK
NTILES = 32
PT = E // NTILES
CH = 80
SCH = 2000
NSCH = PT // SCH
CPS = SCH // CH
NP = 10112
RPT = NP // 16

_NEG_HUGE = -3.4e38


def _proj_tail(feat, al_ref, ar_ref, fe_ref, el_ref, er_ref, elm_ref, erm_ref, i):
    fe_ref[...] = feat
    el = jnp.sum(feat * al_ref[...], axis=1, keepdims=True)
    er = jnp.sum(feat * ar_ref[...], axis=1, keepdims=True)
    el_ref[...] = el
    er_ref[...] = er

    @pl.when(i == 0)
    def _():
        elm_ref[0, 0] = _NEG_HUGE
        erm_ref[0, 0] = _NEG_HUGE

    elm_ref[0, 0] = jnp.maximum(elm_ref[0, 0], jnp.max(el))
    erm_ref[0, 0] = jnp.maximum(erm_ref[0, 0], jnp.max(er))


def _tc1_body(x_ref, w_ref, al_ref, ar_ref, fe_ref, el_ref, er_ref, elm_ref, erm_ref):
    i = pl.program_id(0)
    feat = jnp.dot(x_ref[...], w_ref[...], preferred_element_type=jnp.float32)
    _proj_tail(feat, al_ref, ar_ref, fe_ref, el_ref, er_ref, elm_ref, erm_ref, i)


_proj_outs = dict(
    out_specs=[
        pl.BlockSpec((BLK, D), lambda i: (i, 0)),
        pl.BlockSpec((BLK, 1), lambda i: (i, 0)),
        pl.BlockSpec((BLK, 1), lambda i: (i, 0)),
        pl.BlockSpec(memory_space=pltpu.SMEM),
        pl.BlockSpec(memory_space=pltpu.SMEM),
    ],
    out_shape=[
        jax.ShapeDtypeStruct((N, D), jnp.float32),
        jax.ShapeDtypeStruct((N, 1), jnp.float32),
        jax.ShapeDtypeStruct((N, 1), jnp.float32),
        jax.ShapeDtypeStruct((1, 1), jnp.float32),
        jax.ShapeDtypeStruct((1, 1), jnp.float32),
    ],
)

_tc1 = pl.pallas_call(
    _tc1_body,
    grid=(GRID,),
    in_specs=[
        pl.BlockSpec((BLK, D), lambda i: (i, 0)),
        pl.BlockSpec((D, D), lambda i: (0, 0)),
        pl.BlockSpec((1, D), lambda i: (0, 0)),
        pl.BlockSpec((1, D), lambda i: (0, 0)),
    ],
    **_proj_outs,
)


def _tc2_body(acc_ref, den_ref, b_ref, w_ref, al_ref, ar_ref,
              fe_ref, el_ref, er_ref, elm_ref, erm_ref):
    i = pl.program_id(0)
    accs = acc_ref[0] + acc_ref[1]
    den = den_ref[...]
    rst = jnp.where(den > 0, accs / den, 0.0) + b_ref[...]
    h = jnp.where(rst > 0, rst, jnp.exp(jnp.minimum(rst, 0.0)) - 1.0)
    feat = jnp.dot(h, w_ref[...], preferred_element_type=jnp.float32)
    _proj_tail(feat, al_ref, ar_ref, fe_ref, el_ref, er_ref, elm_ref, erm_ref, i)


_tc2 = pl.pallas_call(
    _tc2_body,
    grid=(GRID,),
    in_specs=[
        pl.BlockSpec((2, BLK, D), lambda i: (0, i, 0)),
        pl.BlockSpec((BLK, 1), lambda i: (i, 0)),
        pl.BlockSpec((1, D), lambda i: (0, 0)),
        pl.BlockSpec((D, D), lambda i: (0, 0)),
        pl.BlockSpec((1, D), lambda i: (0, 0)),
        pl.BlockSpec((1, D), lambda i: (0, 0)),
    ],
    **_proj_outs,
)


def _tc3_body(acc_ref, den_ref, b_ref, fcw_ref, fcb_ref, y_ref, colsum):
    i = pl.program_id(0)
    accs = acc_ref[0] + acc_ref[1]
    den = den_ref[...]
    rst = jnp.where(den > 0, accs / den, 0.0) + b_ref[...]

    @pl.when(i == 0)
    def _():
        colsum[...] = jnp.zeros((1, D), jnp.float32)

    colsum[...] = colsum[...] + jnp.sum(rst, axis=0, keepdims=True)

    @pl.when(i == pl.num_programs(0) - 1)
    def _():
        hg = colsum[...] * jnp.float32(1.0 / N)
        y = jnp.sum(hg * fcw_ref[...], axis=1, keepdims=True) + fcb_ref[0, 0]
        y_ref[...] = 1.0 / (1.0 + jnp.exp(-y))


_tc3 = pl.pallas_call(
    _tc3_body,
    grid=(GRID,),
    in_specs=[
        pl.BlockSpec((2, BLK, D), lambda i: (0, i, 0)),
        pl.BlockSpec((BLK, 1), lambda i: (i, 0)),
        pl.BlockSpec((1, D), lambda i: (0, 0)),
        pl.BlockSpec((1, D), lambda i: (0, 0)),
        pl.BlockSpec(memory_space=pltpu.SMEM),
    ],
    out_specs=pl.BlockSpec((1, 1), lambda i: (0, 0)),
    out_shape=jax.ShapeDtypeStruct((1, 1), jnp.float32),
    scratch_shapes=[pltpu.VMEM((1, D), jnp.float32)],
)


_sc_mesh = plsc.VectorSubcoreMesh(core_axis_name="c", subcore_axis_name="s")


@functools.partial(
    pl.kernel,
    out_type=(
        jax.ShapeDtypeStruct((2, NP, D), jnp.float32),
        jax.ShapeDtypeStruct((NTILES, N), jnp.float32),
    ),
    mesh=_sc_mesh,
    compiler_params=pltpu.CompilerParams(needs_layout_passes=False),
    scratch_types=[
        pltpu.VMEM((SCH,), jnp.int32),
        pltpu.VMEM((SCH,), jnp.int32),
        pltpu.VMEM((N,), jnp.float32),
        pltpu.VMEM((N,), jnp.float32),
        pltpu.VMEM((N,), jnp.float32),
        pltpu.VMEM((CH,), jnp.float32),
        pltpu.VMEM((CH, D), jnp.float32),
        pltpu.VMEM((16,), jnp.float32),
        pltpu.VMEM_SHARED((NP, D), jnp.float32),
        pltpu.SemaphoreType.DMA,
    ],
)
def _sc_edge(feat_hbm, el_hbm, er_hbm, src_hbm, dst_hbm, shift_hbm,
             zeros_hbm, acc_out, den_out,
             src_v, dst_v, el_v, er_v, den_v, ex_v, rows_v, sh_v, acc_sh, sem):
    c = lax.axis_index("c")
    s = lax.axis_index("s")
    wid = c * 16 + s

    pltpu.sync_copy(zeros_hbm, acc_sh.at[pl.ds(s * RPT, RPT)])

    pltpu.sync_copy(el_hbm, el_v)
    pltpu.sync_copy(er_hbm, er_v)
    pltpu.sync_copy(shift_hbm, sh_v)
    shift = sh_v[...]

    def zero_body(g, carry):
        den_v[pl.ds(g * 16, 16)] = jnp.zeros((16,), jnp.float32)
        return carry

    lax.fori_loop(0, N // 16, zero_body, None)

    plsc.subcore_barrier()

    base = wid * PT

    def sch_body(t, carry):
        pltpu.sync_copy(src_hbm.at[pl.ds(base + t * SCH, SCH)], src_v)
        pltpu.sync_copy(dst_hbm.at[pl.ds(base + t * SCH, SCH)], dst_v)

        def chunk_body(j, carry2):
            off = j * CH
            src_sl = src_v.at[pl.ds(off, CH)]
            dst_sl = dst_v.at[pl.ds(off, CH)]
            cp = pltpu.async_copy(feat_hbm.at[src_sl], rows_v, sem)

            def ex_body(g, carry3):
                s16 = src_v[pl.ds(off + g * 16, 16)]
                d16 = dst_v[pl.ds(off + g * 16, 16)]
                z = plsc.load_gather(el_v, [s16]) + plsc.load_gather(er_v, [d16])
                ex = jnp.exp(jnp.maximum(z, 0.2 * z) - shift)
                ex_v[pl.ds(g * 16, 16)] = ex
                plsc.addupdate_scatter(den_v, [d16], ex)
                return carry3

            lax.fori_loop(0, CH // 16, ex_body, None)
            cp.wait()

            def edge_body(k, carry3):
                idx = jnp.zeros((16,), jnp.int32) + k
                exk = plsc.load_gather(ex_v, [idx])
                for col in range(D // 16):
                    sl = pl.ds(col * 16, 16)
                    rows_v[k, sl] = rows_v[k, sl] * exk
                return carry3

            lax.fori_loop(0, CH, edge_body, None)
            pltpu.sync_copy(rows_v, acc_sh.at[dst_sl], add=True)
            return carry2

        lax.fori_loop(0, CPS, chunk_body, None)
        return carry

    lax.fori_loop(0, NSCH, sch_body, None)

    plsc.subcore_barrier()
    pltpu.sync_copy(acc_sh.at[pl.ds(s * RPT, RPT)],
                    acc_out.at[c, pl.ds(s * RPT, RPT)])
    pltpu.sync_copy(den_v, den_out.at[wid])


def _leaky(x):
    return jnp.where(x >= 0, x, 0.2 * x)


def kernel(features, edge_index, W1, al1, ar1, b1, W2, al2, ar2, b2, fcW, fcb):
    src = edge_index[0].astype(jnp.int32)
    dst = edge_index[1].astype(jnp.int32)
    zeros = jnp.zeros((RPT, D), jnp.float32)

    fe1, el1, er1, elm1, erm1 = _tc1(
        features, W1, al1.reshape(1, D), ar1.reshape(1, D))
    sh1 = jnp.full((16,), _leaky(elm1[0, 0] + erm1[0, 0]), jnp.float32)
    acc1, dpart1 = _sc_edge(fe1, el1.reshape(N), er1.reshape(N), src, dst,
                            sh1, zeros)
    den1 = jnp.sum(dpart1, axis=0).reshape(N, 1)

    fe2, el2, er2, elm2, erm2 = _tc2(
        acc1, den1, b1.reshape(1, D), W2, al2.reshape(1, D), ar2.reshape(1, D))
    sh2 = jnp.full((16,), _leaky(elm2[0, 0] + erm2[0, 0]), jnp.float32)
    acc2, dpart2 = _sc_edge(fe2, el2.reshape(N), er2.reshape(N), src, dst,
                            sh2, zeros)
    den2 = jnp.sum(dpart2, axis=0).reshape(N, 1)

    y = _tc3(acc2, den2, b2.reshape(1, D), fcW.reshape(1, D), fcb.reshape(1, 1))
    return y


# speedup vs baseline: 50.0418x; 1.8492x over previous
import functools

import jax
import jax.numpy as jnp
from jax import lax
from jax.experimental import pallas as pl
from jax.experimental.pallas import tpu as pltpu
from jax.experimental.pallas import tpu_sc as plsc

N = 10000
E = 320000
D = 128
BLK = 2000
GRID = N // BLK
NTILES = 32
PT = E // NTILES
CH = 80
SCH = 2000
NSCH = PT // SCH
CPS = SCH // CH
NP = 10112
RPT = NP // 16

_NEG_HUGE = -3.4e38


def _proj_tail(feat, al_ref, ar_ref, fe_ref, elr_ref, elm_ref, erm_ref, i):
    fe_ref[...] = feat
    el = jnp.sum(feat * al_ref[...], axis=1, keepdims=True)
    er = jnp.sum(feat * ar_ref[...], axis=1, keepdims=True)
    el16 = jax.lax.bitcast_convert_type(el.astype(jnp.bfloat16), jnp.uint16)
    er16 = jax.lax.bitcast_convert_type(er.astype(jnp.bfloat16), jnp.uint16)
    packed = (el16.astype(jnp.uint32) << 16) | er16.astype(jnp.uint32)
    elr_ref[...] = jax.lax.bitcast_convert_type(packed, jnp.int32)

    @pl.when(i == 0)
    def _():
        elm_ref[0, 0] = _NEG_HUGE
        erm_ref[0, 0] = _NEG_HUGE

    elm_ref[0, 0] = jnp.maximum(elm_ref[0, 0], jnp.max(el))
    erm_ref[0, 0] = jnp.maximum(erm_ref[0, 0], jnp.max(er))


def _tc1_body(x_ref, w_ref, al_ref, ar_ref, fe_ref, elr_ref, elm_ref, erm_ref):
    i = pl.program_id(0)
    feat = jnp.dot(x_ref[...], w_ref[...], preferred_element_type=jnp.float32)
    _proj_tail(feat, al_ref, ar_ref, fe_ref, elr_ref, elm_ref, erm_ref, i)


_proj_outs = dict(
    out_specs=[
        pl.BlockSpec((BLK, D), lambda i: (i, 0)),
        pl.BlockSpec((BLK, 1), lambda i: (i, 0)),
        pl.BlockSpec(memory_space=pltpu.SMEM),
        pl.BlockSpec(memory_space=pltpu.SMEM),
    ],
    out_shape=[
        jax.ShapeDtypeStruct((N, D), jnp.float32),
        jax.ShapeDtypeStruct((N, 1), jnp.int32),
        jax.ShapeDtypeStruct((1, 1), jnp.float32),
        jax.ShapeDtypeStruct((1, 1), jnp.float32),
    ],
)

_tc1 = pl.pallas_call(
    _tc1_body,
    grid=(GRID,),
    in_specs=[
        pl.BlockSpec((BLK, D), lambda i: (i, 0)),
        pl.BlockSpec((D, D), lambda i: (0, 0)),
        pl.BlockSpec((1, D), lambda i: (0, 0)),
        pl.BlockSpec((1, D), lambda i: (0, 0)),
    ],
    **_proj_outs,
)


def _tc2_body(acc_ref, den_ref, b_ref, w_ref, al_ref, ar_ref,
              fe_ref, elr_ref, elm_ref, erm_ref):
    i = pl.program_id(0)
    accs = acc_ref[0] + acc_ref[1]
    den = den_ref[...]
    rst = jnp.where(den > 0, accs / den, 0.0) + b_ref[...]
    h = jnp.where(rst > 0, rst, jnp.exp(jnp.minimum(rst, 0.0)) - 1.0)
    feat = jnp.dot(h, w_ref[...], preferred_element_type=jnp.float32)
    _proj_tail(feat, al_ref, ar_ref, fe_ref, elr_ref, elm_ref, erm_ref, i)


_tc2 = pl.pallas_call(
    _tc2_body,
    grid=(GRID,),
    in_specs=[
        pl.BlockSpec((2, BLK, D), lambda i: (0, i, 0)),
        pl.BlockSpec((BLK, 1), lambda i: (i, 0)),
        pl.BlockSpec((1, D), lambda i: (0, 0)),
        pl.BlockSpec((D, D), lambda i: (0, 0)),
        pl.BlockSpec((1, D), lambda i: (0, 0)),
        pl.BlockSpec((1, D), lambda i: (0, 0)),
    ],
    **_proj_outs,
)


def _tc3_body(acc_ref, den_ref, b_ref, fcw_ref, fcb_ref, y_ref, colsum):
    i = pl.program_id(0)
    accs = acc_ref[0] + acc_ref[1]
    den = den_ref[...]
    rst = jnp.where(den > 0, accs / den, 0.0) + b_ref[...]

    @pl.when(i == 0)
    def _():
        colsum[...] = jnp.zeros((1, D), jnp.float32)

    colsum[...] = colsum[...] + jnp.sum(rst, axis=0, keepdims=True)

    @pl.when(i == pl.num_programs(0) - 1)
    def _():
        hg = colsum[...] * jnp.float32(1.0 / N)
        y = jnp.sum(hg * fcw_ref[...], axis=1, keepdims=True) + fcb_ref[0, 0]
        y_ref[...] = 1.0 / (1.0 + jnp.exp(-y))


_tc3 = pl.pallas_call(
    _tc3_body,
    grid=(GRID,),
    in_specs=[
        pl.BlockSpec((2, BLK, D), lambda i: (0, i, 0)),
        pl.BlockSpec((BLK, 1), lambda i: (i, 0)),
        pl.BlockSpec((1, D), lambda i: (0, 0)),
        pl.BlockSpec((1, D), lambda i: (0, 0)),
        pl.BlockSpec(memory_space=pltpu.SMEM),
    ],
    out_specs=pl.BlockSpec((1, 1), lambda i: (0, 0)),
    out_shape=jax.ShapeDtypeStruct((1, 1), jnp.float32),
    scratch_shapes=[pltpu.VMEM((1, D), jnp.float32)],
)


_sc_mesh = plsc.VectorSubcoreMesh(core_axis_name="c", subcore_axis_name="s")


@functools.partial(
    pl.kernel,
    out_type=(
        jax.ShapeDtypeStruct((2, NP, D), jnp.float32),
        jax.ShapeDtypeStruct((NTILES, N), jnp.float32),
    ),
    mesh=_sc_mesh,
    compiler_params=pltpu.CompilerParams(needs_layout_passes=False),
    scratch_types=[
        pltpu.VMEM((SCH,), jnp.int32),
        pltpu.VMEM((SCH,), jnp.int32),
        pltpu.VMEM((N,), jnp.int32),
        pltpu.VMEM((N,), jnp.float32),
        pltpu.VMEM((2, CH), jnp.float32),
        pltpu.VMEM((2, CH, D), jnp.float32),
        pltpu.VMEM((16,), jnp.float32),
        pltpu.VMEM_SHARED((NP, D), jnp.float32),
        pltpu.SemaphoreType.DMA,
        pltpu.SemaphoreType.DMA,
        pltpu.SemaphoreType.DMA,
    ],
)
def _sc_edge(feat_hbm, elr_hbm, src_hbm, dst_hbm, shift_hbm,
             zeros_hbm, acc_out, den_out,
             src_v, dst_v, elr_v, den_v, ex_v, rows_v, sh_v, acc_sh,
             sem0, sem1, semz):
    c = lax.axis_index("c")
    s = lax.axis_index("s")
    wid = c * 16 + s

    acc_zero_sl = acc_sh.at[pl.ds(s * RPT, RPT)]
    pltpu.async_copy(zeros_hbm, acc_zero_sl, semz)

    pltpu.sync_copy(elr_hbm, elr_v)
    pltpu.sync_copy(shift_hbm, sh_v)
    shift = sh_v[...]

    def zero_body(g, carry):
        den_v[pl.ds(g * 16, 16)] = jnp.zeros((16,), jnp.float32)
        return carry

    lax.fori_loop(0, N // 16, zero_body, None)

    pltpu.make_async_copy(zeros_hbm, acc_zero_sl, semz).wait()
    plsc.subcore_barrier()

    base = wid * PT
    sems = (sem0, sem1)

    def start_gather(j, b):
        src_sl = src_v.at[pl.ds(j * CH, CH)]
        pltpu.async_copy(feat_hbm.at[src_sl], rows_v.at[b], sems[b])

    def wait_gather(j, b):
        src_sl = src_v.at[pl.ds(j * CH, CH)]
        pltpu.make_async_copy(feat_hbm.at[src_sl], rows_v.at[b], sems[b]).wait()

    def compute_ex(j, b):
        off = j * CH
        for g in range(CH // 16):
            s16 = src_v[pl.ds(off + g * 16, 16)]
            d16 = dst_v[pl.ds(off + g * 16, 16)]
            ps = plsc.load_gather(elr_v, [s16])
            pd = plsc.load_gather(elr_v, [d16])
            el_s = plsc.bitcast(ps & jnp.int32(-65536), jnp.float32)
            er_d = plsc.bitcast(pd << 16, jnp.float32)
            z = el_s + er_d
            ex = jnp.exp(jnp.maximum(z, 0.2 * z) - shift)
            ex_v[b, pl.ds(g * 16, 16)] = ex
            plsc.addupdate_scatter(den_v, [d16], ex)

    def scale_scatter(j, b):
        def group_body(g, carry):
            ex16 = ex_v[b, pl.ds(g * 16, 16)]
            for kk in range(16):
                k = g * 16 + kk
                e = ex16[kk]
                for col in range(D // 16):
                    sl = pl.ds(col * 16, 16)
                    rows_v[b, k, sl] = rows_v[b, k, sl] * e
            return carry

        lax.fori_loop(0, CH // 16, group_body, None)
        dst_sl = dst_v.at[pl.ds(j * CH, CH)]
        pltpu.sync_copy(rows_v.at[b], acc_sh.at[dst_sl], add=True)

    def chunk_iter(j, b, prefetch):
        if prefetch:
            start_gather(j + 1, 1 - b)
            compute_ex(j + 1, 1 - b)
        wait_gather(j, b)
        scale_scatter(j, b)

    def sch_body(t, carry):
        pltpu.sync_copy(src_hbm.at[pl.ds(base + t * SCH, SCH)], src_v)
        pltpu.sync_copy(dst_hbm.at[pl.ds(base + t * SCH, SCH)], dst_v)

        start_gather(0, 0)
        compute_ex(0, 0)

        def pair_body(p, carry2):
            chunk_iter(2 * p, 0, True)
            chunk_iter(2 * p + 1, 1, True)
            return carry2

        lax.fori_loop(0, (CPS - 1) // 2, pair_body, None)
        chunk_iter(CPS - 1, 0, False)
        return carry

    lax.fori_loop(0, NSCH, sch_body, None)

    plsc.subcore_barrier()
    pltpu.sync_copy(acc_sh.at[pl.ds(s * RPT, RPT)],
                    acc_out.at[c, pl.ds(s * RPT, RPT)])
    pltpu.sync_copy(den_v, den_out.at[wid])


def _leaky(x):
    return jnp.where(x >= 0, x, 0.2 * x)


def kernel(features, edge_index, W1, al1, ar1, b1, W2, al2, ar2, b2, fcW, fcb):
    src = edge_index[0].astype(jnp.int32)
    dst = edge_index[1].astype(jnp.int32)
    zeros = jnp.zeros((RPT, D), jnp.float32)

    fe1, elr1, elm1, erm1 = _tc1(
        features, W1, al1.reshape(1, D), ar1.reshape(1, D))
    sh1 = jnp.full((16,), _leaky(elm1[0, 0] + erm1[0, 0]), jnp.float32)
    acc1, dpart1 = _sc_edge(fe1, elr1.reshape(N), src, dst, sh1, zeros)
    den1 = jnp.sum(dpart1, axis=0).reshape(N, 1)

    fe2, elr2, elm2, erm2 = _tc2(
        acc1, den1, b1.reshape(1, D), W2, al2.reshape(1, D), ar2.reshape(1, D))
    sh2 = jnp.full((16,), _leaky(elm2[0, 0] + erm2[0, 0]), jnp.float32)
    acc2, dpart2 = _sc_edge(fe2, elr2.reshape(N), src, dst, sh2, zeros)
    den2 = jnp.sum(dpart2, axis=0).reshape(N, 1)

    y = _tc3(acc2, den2, b2.reshape(1, D), fcW.reshape(1, D), fcb.reshape(1, 1))
    return y


# speedup vs baseline: 54.7193x; 1.0935x over previous
import functools

import jax
import jax.numpy as jnp
from jax import lax
from jax.experimental import pallas as pl
from jax.experimental.pallas import tpu as pltpu
from jax.experimental.pallas import tpu_sc as plsc

N = 10000
E = 320000
D = 128
BLK = 2000
GRID = N // BLK
NTILES = 32
PT = E // NTILES
CH = 80
SCH = 2000
NSCH = PT // SCH
CPS = SCH // CH
NP = 10240
RPT = NP // 16
NBUF = 3

_NEG_HUGE = -3.4e38


def _proj_tail(feat, al_ref, ar_ref, fe_ref, elr_ref, elm_ref, erm_ref, i):
    fe_ref[...] = feat
    el = jnp.sum(feat * al_ref[...], axis=1, keepdims=True)
    er = jnp.sum(feat * ar_ref[...], axis=1, keepdims=True)
    el16 = jax.lax.bitcast_convert_type(el.astype(jnp.bfloat16), jnp.uint16)
    er16 = jax.lax.bitcast_convert_type(er.astype(jnp.bfloat16), jnp.uint16)
    packed = (el16.astype(jnp.uint32) << 16) | er16.astype(jnp.uint32)
    elr_ref[...] = jax.lax.bitcast_convert_type(packed, jnp.int32)

    @pl.when(i == 0)
    def _():
        elm_ref[0, 0] = _NEG_HUGE
        erm_ref[0, 0] = _NEG_HUGE

    elm_ref[0, 0] = jnp.maximum(elm_ref[0, 0], jnp.max(el))
    erm_ref[0, 0] = jnp.maximum(erm_ref[0, 0], jnp.max(er))


def _tc1_body(x_ref, w_ref, al_ref, ar_ref, fe_ref, elr_ref, elm_ref, erm_ref):
    i = pl.program_id(0)
    feat = jnp.dot(x_ref[...], w_ref[...], preferred_element_type=jnp.float32)
    _proj_tail(feat, al_ref, ar_ref, fe_ref, elr_ref, elm_ref, erm_ref, i)


_proj_outs = dict(
    out_specs=[
        pl.BlockSpec((BLK, D), lambda i: (i, 0)),
        pl.BlockSpec((BLK, 1), lambda i: (i, 0)),
        pl.BlockSpec(memory_space=pltpu.SMEM),
        pl.BlockSpec(memory_space=pltpu.SMEM),
    ],
    out_shape=[
        jax.ShapeDtypeStruct((N, D), jnp.float32),
        jax.ShapeDtypeStruct((N, 1), jnp.int32),
        jax.ShapeDtypeStruct((1, 1), jnp.float32),
        jax.ShapeDtypeStruct((1, 1), jnp.float32),
    ],
)

_tc1 = pl.pallas_call(
    _tc1_body,
    grid=(GRID,),
    in_specs=[
        pl.BlockSpec((BLK, D), lambda i: (i, 0)),
        pl.BlockSpec((D, D), lambda i: (0, 0)),
        pl.BlockSpec((1, D), lambda i: (0, 0)),
        pl.BlockSpec((1, D), lambda i: (0, 0)),
    ],
    **_proj_outs,
)


def _tc2_body(acc_ref, den_ref, b_ref, w_ref, al_ref, ar_ref,
              fe_ref, elr_ref, elm_ref, erm_ref):
    i = pl.program_id(0)
    accs = acc_ref[0] + acc_ref[1]
    den = den_ref[...]
    rst = jnp.where(den > 0, accs / den, 0.0) + b_ref[...]
    h = jnp.where(rst > 0, rst, jnp.exp(jnp.minimum(rst, 0.0)) - 1.0)
    feat = jnp.dot(h, w_ref[...], preferred_element_type=jnp.float32)
    _proj_tail(feat, al_ref, ar_ref, fe_ref, elr_ref, elm_ref, erm_ref, i)


_tc2 = pl.pallas_call(
    _tc2_body,
    grid=(GRID,),
    in_specs=[
        pl.BlockSpec((2, BLK, D), lambda i: (0, i, 0)),
        pl.BlockSpec((BLK, 1), lambda i: (i, 0)),
        pl.BlockSpec((1, D), lambda i: (0, 0)),
        pl.BlockSpec((D, D), lambda i: (0, 0)),
        pl.BlockSpec((1, D), lambda i: (0, 0)),
        pl.BlockSpec((1, D), lambda i: (0, 0)),
    ],
    **_proj_outs,
)


def _tc3_body(acc_ref, den_ref, b_ref, fcw_ref, fcb_ref, y_ref, colsum):
    i = pl.program_id(0)
    accs = acc_ref[0] + acc_ref[1]
    den = den_ref[...]
    rst = jnp.where(den > 0, accs / den, 0.0) + b_ref[...]

    @pl.when(i == 0)
    def _():
        colsum[...] = jnp.zeros((1, D), jnp.float32)

    colsum[...] = colsum[...] + jnp.sum(rst, axis=0, keepdims=True)

    @pl.when(i == pl.num_programs(0) - 1)
    def _():
        hg = colsum[...] * jnp.float32(1.0 / N)
        y = jnp.sum(hg * fcw_ref[...], axis=1, keepdims=True) + fcb_ref[0, 0]
        y_ref[...] = 1.0 / (1.0 + jnp.exp(-y))


_tc3 = pl.pallas_call(
    _tc3_body,
    grid=(GRID,),
    in_specs=[
        pl.BlockSpec((2, BLK, D), lambda i: (0, i, 0)),
        pl.BlockSpec((BLK, 1), lambda i: (i, 0)),
        pl.BlockSpec((1, D), lambda i: (0, 0)),
        pl.BlockSpec((1, D), lambda i: (0, 0)),
        pl.BlockSpec(memory_space=pltpu.SMEM),
    ],
    out_specs=pl.BlockSpec((1, 1), lambda i: (0, 0)),
    out_shape=jax.ShapeDtypeStruct((1, 1), jnp.float32),
    scratch_shapes=[pltpu.VMEM((1, D), jnp.float32)],
)


_sc_mesh = plsc.VectorSubcoreMesh(core_axis_name="c", subcore_axis_name="s")


@functools.partial(
    pl.kernel,
    out_type=(
        jax.ShapeDtypeStruct((2, NP, D), jnp.float32),
        jax.ShapeDtypeStruct((2, NP), jnp.float32),
    ),
    mesh=_sc_mesh,
    compiler_params=pltpu.CompilerParams(needs_layout_passes=False),
    scratch_types=[
        pltpu.VMEM((SCH,), jnp.int32),
        pltpu.VMEM((SCH,), jnp.int32),
        pltpu.VMEM((N,), jnp.int32),
        pltpu.VMEM((NBUF, CH), jnp.float32),
        pltpu.VMEM((NBUF, CH, D), jnp.float32),
        pltpu.VMEM((16,), jnp.float32),
        pltpu.VMEM_SHARED((NP, D), jnp.float32),
        pltpu.VMEM_SHARED((NP,), jnp.float32),
        pltpu.SemaphoreType.DMA,
        pltpu.SemaphoreType.DMA,
        pltpu.SemaphoreType.DMA,
        pltpu.SemaphoreType.DMA,
        pltpu.SemaphoreType.DMA,
        pltpu.SemaphoreType.DMA,
        pltpu.SemaphoreType.DMA,
        pltpu.SemaphoreType.DMA,
    ],
)
def _sc_edge(feat_hbm, elr_hbm, src_hbm, dst_hbm, shift_hbm,
             zeros_hbm, zerosd_hbm, acc_out, den_out,
             src_v, dst_v, elr_v, ex_v, rows_v, sh_v, acc_sh, den_sh,
             semg0, semg1, semg2, sems0, sems1, sems2, semz, semzd):
    c = lax.axis_index("c")
    s = lax.axis_index("s")

    acc_zero_sl = acc_sh.at[pl.ds(s * RPT, RPT)]
    pltpu.async_copy(zeros_hbm, acc_zero_sl, semz)
    den_zero_sl = den_sh.at[pl.ds(s * RPT, RPT)]
    pltpu.async_copy(zerosd_hbm, den_zero_sl, semzd)

    pltpu.sync_copy(elr_hbm, elr_v)
    pltpu.sync_copy(shift_hbm, sh_v)
    shift = sh_v[...]

    pltpu.make_async_copy(zerosd_hbm, den_zero_sl, semzd).wait()
    pltpu.make_async_copy(zeros_hbm, acc_zero_sl, semz).wait()
    plsc.subcore_barrier()

    base = (c * 16 + s) * PT
    sems_g = (semg0, semg1, semg2)
    sems_s = (sems0, sems1, sems2)

    def start_gather(j, b):
        src_sl = src_v.at[pl.ds(j * CH, CH)]
        pltpu.async_copy(feat_hbm.at[src_sl], rows_v.at[b], sems_g[b])

    def wait_gather(j, b):
        src_sl = src_v.at[pl.ds(j * CH, CH)]
        pltpu.make_async_copy(
            feat_hbm.at[src_sl], rows_v.at[b], sems_g[b]).wait()

    def compute_ex(j, b):
        off = j * CH
        for g in range(CH // 16):
            s16 = src_v[pl.ds(off + g * 16, 16)]
            d16 = dst_v[pl.ds(off + g * 16, 16)]
            ps = plsc.load_gather(elr_v, [s16])
            pd = plsc.load_gather(elr_v, [d16])
            el_s = plsc.bitcast(ps & jnp.int32(-65536), jnp.float32)
            er_d = plsc.bitcast(pd << 16, jnp.float32)
            z = el_s + er_d
            ex = jnp.exp(jnp.maximum(z, 0.2 * z) - shift)
            ex_v[b, pl.ds(g * 16, 16)] = ex
        dst_sl = dst_v.at[pl.ds(off, CH)]
        pltpu.sync_copy(ex_v.at[b], den_sh.at[dst_sl], add=True)

    def start_scatter(j, b):
        dst_sl = dst_v.at[pl.ds(j * CH, CH)]
        pltpu.async_copy(rows_v.at[b], acc_sh.at[dst_sl], sems_s[b],
                         add=True)

    def wait_scatter(j, b):
        dst_sl = dst_v.at[pl.ds(j * CH, CH)]
        pltpu.make_async_copy(rows_v.at[b], acc_sh.at[dst_sl],
                              sems_s[b]).wait()

    def scale(j, b):
        def group_body(g, carry):
            ex16 = ex_v[b, pl.ds(g * 16, 16)]
            for kk in range(16):
                k = g * 16 + kk
                e = ex16[kk]
                for col in range(D // 16):
                    sl = pl.ds(col * 16, 16)
                    rows_v[b, k, sl] = rows_v[b, k, sl] * e
            return carry

        lax.fori_loop(0, CH // 16, group_body, None)

    def chunk_iter(j, b, prefetch, wait_sc):
        if prefetch:
            bn = (b + 1) % NBUF
            if wait_sc:
                wait_scatter(j + 1 - NBUF, bn)
            start_gather(j + 1, bn)
            compute_ex(j + 1, bn)
        wait_gather(j, b)
        scale(j, b)
        start_scatter(j, b)

    def sch_body(t, carry):
        pltpu.sync_copy(src_hbm.at[pl.ds(base + t * SCH, SCH)], src_v)
        pltpu.sync_copy(dst_hbm.at[pl.ds(base + t * SCH, SCH)], dst_v)

        start_gather(0, 0)
        compute_ex(0, 0)
        chunk_iter(0, 0, True, False)
        chunk_iter(1, 1, True, False)

        def triple_body(p, carry2):
            j0 = 2 + 3 * p
            chunk_iter(j0, 2, True, True)
            chunk_iter(j0 + 1, 0, True, True)
            chunk_iter(j0 + 2, 1, True, True)
            return carry2

        lax.fori_loop(0, (CPS - 4) // 3, triple_body, None)
        chunk_iter(CPS - 2, 2, True, True)
        chunk_iter(CPS - 1, 0, False, False)
        wait_scatter(CPS - 3, 1)
        wait_scatter(CPS - 2, 2)
        wait_scatter(CPS - 1, 0)
        return carry

    lax.fori_loop(0, NSCH, sch_body, None)

    plsc.subcore_barrier()
    pltpu.sync_copy(acc_sh.at[pl.ds(s * RPT, RPT)],
                    acc_out.at[c, pl.ds(s * RPT, RPT)])
    pltpu.sync_copy(den_sh.at[pl.ds(s * RPT, RPT)],
                    den_out.at[c, pl.ds(s * RPT, RPT)])


def _leaky(x):
    return jnp.where(x >= 0, x, 0.2 * x)


def kernel(features, edge_index, W1, al1, ar1, b1, W2, al2, ar2, b2, fcW, fcb):
    src = edge_index[0].astype(jnp.int32)
    dst = edge_index[1].astype(jnp.int32)
    zeros = jnp.zeros((RPT, D), jnp.float32)
    zerosd = jnp.zeros((RPT,), jnp.float32)

    fe1, elr1, elm1, erm1 = _tc1(
        features, W1, al1.reshape(1, D), ar1.reshape(1, D))
    sh1 = jnp.full((16,), _leaky(elm1[0, 0] + erm1[0, 0]), jnp.float32)
    acc1, dpart1 = _sc_edge(fe1, elr1.reshape(N), src, dst, sh1, zeros, zerosd)
    den1 = jnp.sum(dpart1, axis=0)[:N].reshape(N, 1)

    fe2, elr2, elm2, erm2 = _tc2(
        acc1, den1, b1.reshape(1, D), W2, al2.reshape(1, D), ar2.reshape(1, D))
    sh2 = jnp.full((16,), _leaky(elm2[0, 0] + erm2[0, 0]), jnp.float32)
    acc2, dpart2 = _sc_edge(fe2, elr2.reshape(N), src, dst, sh2, zeros, zerosd)
    den2 = jnp.sum(dpart2, axis=0)[:N].reshape(N, 1)

    y = _tc3(acc2, den2, b2.reshape(1, D), fcW.reshape(1, D), fcb.reshape(1, 1))
    return y
